# Initial kernel scaffold; baseline (speedup 1.0000x reference)
import jax
import jax.numpy as jnp
from jax.experimental import pallas as pl


def kernel(x, Wg, Wn, We, be):
    raise NotImplementedError("write your pallas kernel here")


# speedup vs baseline: 5.5273x; 5.5273x over previous
import jax
import jax.numpy as jnp
from jax.experimental import pallas as pl
from jax.experimental.pallas import tpu as pltpu

BS_ = 2048
D_ = 768
NE_ = 8


def _moe_kernel(x_ref, wg_ref, wn_ref, we_ref, be_ref, out_ref, xbf_ref, wt_ref):
    e = pl.program_id(0)

    @pl.when(e == 0)
    def _prologue():
        x = x_ref[...]
        xbf_ref[...] = x.astype(jnp.bfloat16)
        lg = jax.lax.dot_general(
            wg_ref[...], x, (((1,), (1,)), ((), ())),
            preferred_element_type=jnp.float32)
        ln = jax.lax.dot_general(
            wn_ref[...], x, (((1,), (1,)), ((), ())),
            preferred_element_type=jnp.float32)
        sp = jnp.maximum(ln, 0.0) + jnp.log1p(jnp.exp(-jnp.abs(ln)))
        logits = lg + sp
        m = jnp.max(logits, axis=0, keepdims=True)
        p = jnp.exp(logits - m)
        probs = p / jnp.sum(p, axis=0, keepdims=True)
        idx = jax.lax.broadcasted_iota(jnp.int32, (NE_, BS_), 0)
        m1 = jnp.max(probs, axis=0, keepdims=True)
        a1 = jnp.min(jnp.where(probs == m1, idx, NE_), axis=0, keepdims=True)
        masked = jnp.where(idx == a1, -jnp.inf, probs)
        m2 = jnp.max(masked, axis=0, keepdims=True)
        a2 = jnp.min(jnp.where(masked == m2, idx, NE_), axis=0, keepdims=True)
        i_lo = jnp.minimum(a1, a2)
        i_hi = jnp.maximum(a1, a2)
        w = (jnp.where(idx == i_lo, m1, 0.0)
             + jnp.where(idx == i_hi, m2, 0.0))
        wt_ref[...] = w * (1.0 / BS_)
        out_ref[...] = jnp.zeros_like(out_ref)

    we_bf = we_ref[0].astype(jnp.bfloat16)
    z = jax.lax.dot_general(
        xbf_ref[...], we_bf, (((1,), (1,)), ((), ())),
        preferred_element_type=jnp.float32)
    y = jax.nn.sigmoid(z + be_ref[0])
    part = jax.lax.dot_general(
        wt_ref[pl.ds(e, 1), :], y, (((1,), (0,)), ((), ())),
        preferred_element_type=jnp.float32)
    out_ref[...] += part


def kernel(x, Wg, Wn, We, be):
    out = pl.pallas_call(
        _moe_kernel,
        grid=(NE_,),
        in_specs=[
            pl.BlockSpec((BS_, D_), lambda e: (0, 0)),
            pl.BlockSpec((NE_, D_), lambda e: (0, 0)),
            pl.BlockSpec((NE_, D_), lambda e: (0, 0)),
            pl.BlockSpec((1, D_, D_), lambda e: (e, 0, 0)),
            pl.BlockSpec((1, 1, D_), lambda e: (e, 0, 0)),
        ],
        out_specs=pl.BlockSpec((1, D_), lambda e: (0, 0)),
        out_shape=jax.ShapeDtypeStruct((1, D_), jnp.float32),
        scratch_shapes=[
            pltpu.VMEM((BS_, D_), jnp.bfloat16),
            pltpu.VMEM((NE_, BS_), jnp.float32),
        ],
    )(x, Wg, Wn, We, be.reshape(NE_, 1, D_))
    return out.reshape(D_)


# speedup vs baseline: 5.5681x; 1.0074x over previous
import jax
import jax.numpy as jnp
from jax.experimental import pallas as pl
from jax.experimental.pallas import tpu as pltpu

BS_ = 2048
D_ = 768
NE_ = 8


def _moe_kernel(x_ref, wg_ref, wn_ref, we_ref, be_ref, out_ref, xbf_ref, wt_ref):
    e = pl.program_id(0)

    @pl.when(e == 0)
    def _prologue():
        x = x_ref[...]
        xbf_ref[...] = x.astype(jnp.bfloat16)
        lg = jax.lax.dot_general(
            wg_ref[...], x, (((1,), (1,)), ((), ())),
            preferred_element_type=jnp.float32)
        ln = jax.lax.dot_general(
            wn_ref[...], x, (((1,), (1,)), ((), ())),
            preferred_element_type=jnp.float32)
        sp = jnp.maximum(ln, 0.0) + jnp.log1p(jnp.exp(-jnp.abs(ln)))
        logits = lg + sp
        m = jnp.max(logits, axis=0, keepdims=True)
        p = jnp.exp(logits - m)
        probs = p / jnp.sum(p, axis=0, keepdims=True)
        idx = jax.lax.broadcasted_iota(jnp.int32, (NE_, BS_), 0)
        m1 = jnp.max(probs, axis=0, keepdims=True)
        a1 = jnp.min(jnp.where(probs == m1, idx, NE_), axis=0, keepdims=True)
        masked = jnp.where(idx == a1, -jnp.inf, probs)
        m2 = jnp.max(masked, axis=0, keepdims=True)
        a2 = jnp.min(jnp.where(masked == m2, idx, NE_), axis=0, keepdims=True)
        i_lo = jnp.minimum(a1, a2)
        i_hi = jnp.maximum(a1, a2)
        w = (jnp.where(idx == i_lo, m1, 0.0)
             + jnp.where(idx == i_hi, m2, 0.0))
        wt_ref[...] = w * (1.0 / BS_)
        out_ref[...] = jnp.zeros_like(out_ref)

    we_bf = we_ref[0].astype(jnp.bfloat16)
    z = jax.lax.dot_general(
        xbf_ref[...], we_bf, (((1,), (1,)), ((), ())),
        preferred_element_type=jnp.float32)
    t = jnp.tanh((z + be_ref[0]) * 0.5).astype(jnp.bfloat16)
    wrow = wt_ref[pl.ds(e, 1), :]
    part = jax.lax.dot_general(
        wrow.astype(jnp.bfloat16), t, (((1,), (0,)), ((), ())),
        preferred_element_type=jnp.float32)
    out_ref[...] += 0.5 * part + 0.5 * jnp.sum(wrow)


def kernel(x, Wg, Wn, We, be):
    out = pl.pallas_call(
        _moe_kernel,
        grid=(NE_,),
        in_specs=[
            pl.BlockSpec((BS_, D_), lambda e: (0, 0)),
            pl.BlockSpec((NE_, D_), lambda e: (0, 0)),
            pl.BlockSpec((NE_, D_), lambda e: (0, 0)),
            pl.BlockSpec((1, D_, D_), lambda e: (e, 0, 0)),
            pl.BlockSpec((1, 1, D_), lambda e: (e, 0, 0)),
        ],
        out_specs=pl.BlockSpec((1, D_), lambda e: (0, 0)),
        out_shape=jax.ShapeDtypeStruct((1, D_), jnp.float32),
        scratch_shapes=[
            pltpu.VMEM((BS_, D_), jnp.bfloat16),
            pltpu.VMEM((NE_, BS_), jnp.float32),
        ],
    )(x, Wg, Wn, We, be.reshape(NE_, 1, D_))
    return out.reshape(D_)
